# baseline (speedup 1.0000x reference)
import functools

import jax
import jax.numpy as jnp
from jax import lax
from jax.experimental import pallas as pl
from jax.experimental.pallas import tpu as pltpu
from jax.experimental.pallas import tpu_sc as plsc

_N = 10000
_E = 320000
_DF = 128
_DE = 16
_DM = 128


_TE = 12800


def _hw_body(h_ref, ws_ref, wr_ref, hws_ref, hwr_ref):
    h = h_ref[...]
    hws_ref[...] = jnp.dot(h, ws_ref[...], preferred_element_type=jnp.float32)
    hwr_ref[...] = jnp.dot(h, wr_ref[...], preferred_element_type=jnp.float32)


def _node_proj(h, w_s, w_r):
    return pl.pallas_call(
        _hw_body,
        out_shape=[
            jax.ShapeDtypeStruct((_N, _DM), jnp.float32),
            jax.ShapeDtypeStruct((_N, _DM), jnp.float32),
        ],
    )(h, w_s, w_r)


def _ewb_body(et_ref, we_ref, b_ref, out_ref):
    out_ref[...] = (
        lax.dot_general(et_ref[...], we_ref[...], (((0,), (0,)), ((), ())),
                        preferred_element_type=jnp.float32)
        + b_ref[...]
    )


def _edge_proj(et, w_e, b_msg):
    grid = (_E // _TE,)
    return pl.pallas_call(
        _ewb_body,
        grid=grid,
        in_specs=[
            pl.BlockSpec((_DE, _TE), lambda i: (0, i)),
            pl.BlockSpec((_DE, _DM), lambda i: (0, 0)),
            pl.BlockSpec((1, _DM), lambda i: (0, 0)),
        ],
        out_specs=pl.BlockSpec((_TE, _DM), lambda i: (i, 0)),
        out_shape=jax.ShapeDtypeStruct((_E, _DM), jnp.float32),
    )(et, w_e, b_msg.reshape(1, _DM))


_NC = 2
_NS = 16
_NW = _NC * _NS
_EPW = _E // _NW
_C = 16
_NCH = _EPW // _C
_NP = 10240
_RPS = _NP // _NS
_ZR = 32


def _sc_body(ewb_hbm, hws_hbm, hwr_hbm, snd_hbm, rcv_hbm, msk_hbm,
             m_hbm, agg_hbm,
             snd_v, rcv_v, msk_v, ewb_v, gs_v, gr_v, mout_v,
             mexp_v, z_v, agg_sh, isem, dsem, stsem):
    cid = lax.axis_index("c")
    sid = lax.axis_index("s")
    wid = sid * _NC + cid
    wbase = wid * _EPW

    def zrow(i, _):
        for j in range(_DM // 16):
            z_v[i, pl.ds(j * 16, 16)] = jnp.zeros((16,), jnp.float32)
        return 0

    lax.fori_loop(0, _ZR, zrow, 0)
    for k in range(_RPS // _ZR):
        pltpu.sync_copy(z_v, agg_sh.at[pl.ds(sid * _RPS + k * _ZR, _ZR), :])
    plsc.subcore_barrier()

    def issue_idx(c, s4):
        base = wbase + c * _C
        pltpu.async_copy(snd_hbm.at[pl.ds(base, _C)], snd_v[s4], isem[s4])
        pltpu.async_copy(rcv_hbm.at[pl.ds(base, _C)], rcv_v[s4], isem[s4])

    def drain_idx(s4):
        pltpu.make_async_copy(snd_hbm.at[pl.ds(0, _C)], snd_v[s4], isem[s4]).wait()
        pltpu.make_async_copy(rcv_hbm.at[pl.ds(0, _C)], rcv_v[s4], isem[s4]).wait()

    def issue_data(c, s2, s4):
        base = wbase + c * _C
        pltpu.async_copy(msk_hbm.at[pl.ds(base, _C)], msk_v[s2], dsem[s2])
        pltpu.async_copy(ewb_hbm.at[pl.ds(base, _C), :], ewb_v[s2], dsem[s2])
        pltpu.async_copy(hws_hbm.at[snd_v[s4]], gs_v[s2], dsem[s2])
        pltpu.async_copy(hwr_hbm.at[rcv_v[s4]], gr_v[s2], dsem[s2])

    def drain_data(s2, s4):
        pltpu.make_async_copy(msk_hbm.at[pl.ds(0, _C)], msk_v[s2], dsem[s2]).wait()
        pltpu.make_async_copy(ewb_hbm.at[pl.ds(0, _C), :], ewb_v[s2], dsem[s2]).wait()
        pltpu.make_async_copy(hws_hbm.at[snd_v[s4]], gs_v[s2], dsem[s2]).wait()
        pltpu.make_async_copy(hwr_hbm.at[rcv_v[s4]], gr_v[s2], dsem[s2]).wait()

    def issue_wb(c, s4):
        base = wbase + c * _C
        pltpu.async_copy(mout_v[s4], m_hbm.at[pl.ds(base, _C), :], stsem[s4])
        pltpu.sync_copy(mout_v[s4], agg_sh.at[rcv_v[s4]], add=True)

    def drain_wb(s4):
        pltpu.make_async_copy(mout_v[s4], m_hbm.at[pl.ds(0, _C), :], stsem[s4]).wait()

    def compute(s2, s4):
        mskv = msk_v[s2][:]
        for rs in range(_C):
            mexp_v[pl.ds(rs * 16, 16)] = jnp.take(
                mskv, jnp.full((16,), rs, jnp.int32))

        def row(r, _):
            mv = mexp_v[pl.ds(r * 16, 16)]
            for j in range(_DM // 16):
                sl = pl.ds(j * 16, 16)
                x = ewb_v[s2][r, sl] + gs_v[s2][r, sl] + gr_v[s2][r, sl]
                mout_v[s4][r, sl] = jnp.maximum(x, 0.0) * mv
            return 0

        lax.fori_loop(0, _C, row, 0)

    def step(k, s4, idx_issue=True, data_issue=True, st_drain=True):
        if idx_issue:
            if st_drain:
                drain_wb((s4 + 2) % 4)
            issue_idx(k + 2, (s4 + 2) % 4)
        if data_issue:
            drain_idx((s4 + 1) % 4)
            issue_data(k + 1, (s4 + 1) % 2, (s4 + 1) % 4)
        drain_data(s4 % 2, s4)
        compute(s4 % 2, s4)
        issue_wb(k, s4)

    issue_idx(jnp.int32(0), 0)
    issue_idx(jnp.int32(1), 1)
    drain_idx(0)
    issue_data(jnp.int32(0), 0, 0)
    step(jnp.int32(0), 0, st_drain=False)
    step(jnp.int32(1), 1, st_drain=False)

    def group(i, _):
        k0 = 2 + 4 * i
        for u in range(4):
            step(k0 + u, (2 + u) % 4)
        return 0

    lax.fori_loop(0, (_NCH - 7) // 4, group, 0)

    kt = jnp.int32(_NCH - 7)
    for t in range(7):
        k = kt + t
        s4 = (618 + t) % 4
        step(k, s4,
             idx_issue=(618 + t + 2 <= _NCH - 1),
             data_issue=(618 + t + 1 <= _NCH - 1))
    for c in range(_NCH - 4, _NCH):
        drain_wb(c % 4)

    plsc.subcore_barrier()

    for k in range(_RPS // _ZR):
        r0 = sid * _RPS + k * _ZR
        pltpu.sync_copy(agg_sh.at[pl.ds(r0, _ZR), :],
                        agg_hbm.at[cid, pl.ds(r0, _ZR), :])


@functools.lru_cache(maxsize=1)
def _sc_edge():
    return pl.kernel(
        _sc_body,
        out_type=[
            jax.ShapeDtypeStruct((_E, _DM), jnp.float32),
            jax.ShapeDtypeStruct((_NC, _NP, _DM), jnp.float32),
        ],
        mesh=plsc.VectorSubcoreMesh(
            core_axis_name="c", subcore_axis_name="s",
            num_cores=_NC, num_subcores=_NS,
        ),
        scratch_types=[
            [pltpu.VMEM((_C,), jnp.int32) for _ in range(4)],
            [pltpu.VMEM((_C,), jnp.int32) for _ in range(4)],
            [pltpu.VMEM((_C,), jnp.float32) for _ in range(2)],
            [pltpu.VMEM((_C, _DM), jnp.float32) for _ in range(2)],
            [pltpu.VMEM((_C, _DM), jnp.float32) for _ in range(2)],
            [pltpu.VMEM((_C, _DM), jnp.float32) for _ in range(2)],
            [pltpu.VMEM((_C, _DM), jnp.float32) for _ in range(4)],
            pltpu.VMEM((_C * 16,), jnp.float32),
            pltpu.VMEM((_ZR, _DM), jnp.float32),
            pltpu.VMEM_SHARED((_NP, _DM), jnp.float32),
            [pltpu.SemaphoreType.DMA for _ in range(4)],
            [pltpu.SemaphoreType.DMA for _ in range(2)],
            [pltpu.SemaphoreType.DMA for _ in range(4)],
        ],
    )


_TN = 2000


def _node_body(h_ref, agg_ref, wn1_ref, wn2_ref, b_ref, out_ref):
    agg = agg_ref[0] + agg_ref[1]
    x = (
        jnp.dot(h_ref[...], wn1_ref[...], preferred_element_type=jnp.float32)
        + jnp.dot(agg, wn2_ref[...], preferred_element_type=jnp.float32)
        + b_ref[...]
    )
    out_ref[...] = jnp.maximum(x, 0.0)


def _node_update(h, agg2, wn1, wn2, b_node):
    grid = (_N // _TN,)
    return pl.pallas_call(
        _node_body,
        grid=grid,
        in_specs=[
            pl.BlockSpec((_TN, _DF), lambda i: (i, 0)),
            pl.BlockSpec((_NC, _TN, _DM), lambda i: (0, i, 0)),
            pl.BlockSpec((_DF, _DF), lambda i: (0, 0)),
            pl.BlockSpec((_DM, _DF), lambda i: (0, 0)),
            pl.BlockSpec((1, _DF), lambda i: (0, 0)),
        ],
        out_specs=pl.BlockSpec((_TN, _DF), lambda i: (i, 0)),
        out_shape=jax.ShapeDtypeStruct((_N, _DF), jnp.float32),
    )(h, agg2, wn1, wn2, b_node.reshape(1, _DF))


def _edge_body(et_ref, m_ref, we1_ref, we2_ref, b_ref, out_ref):
    x = (
        lax.dot_general(we1_ref[...], et_ref[...], (((0,), (0,)), ((), ())),
                        preferred_element_type=jnp.float32)
        + lax.dot_general(we2_ref[...], m_ref[...], (((0,), (1,)), ((), ())),
                          preferred_element_type=jnp.float32)
        + b_ref[...]
    )
    out_ref[...] = jnp.maximum(x, 0.0)


def _edge_update(et, m, we1, we2, b_edge):
    grid = (_E // _TE,)
    return pl.pallas_call(
        _edge_body,
        grid=grid,
        in_specs=[
            pl.BlockSpec((_DE, _TE), lambda i: (0, i)),
            pl.BlockSpec((_TE, _DM), lambda i: (i, 0)),
            pl.BlockSpec((_DE, _DE), lambda i: (0, 0)),
            pl.BlockSpec((_DM, _DE), lambda i: (0, 0)),
            pl.BlockSpec((_DE, 1), lambda i: (0, 0)),
        ],
        out_specs=pl.BlockSpec((_DE, _TE), lambda i: (0, i)),
        out_shape=jax.ShapeDtypeStruct((_DE, _E), jnp.float32),
    )(et, m, we1, we2, b_edge.reshape(_DE, 1))


@jax.jit
def kernel(h, e, senders, receivers, edge_mask,
           W_msg, b_msg, W_node, b_node, W_edge, b_edge):
    w_e = W_msg[:_DE]
    w_s = W_msg[_DE:_DE + _DF]
    w_r = W_msg[_DE + _DF:]

    et = e.T

    hws, hwr = _node_proj(h, w_s, w_r)
    ewb = _edge_proj(et, w_e, b_msg)

    m, agg2 = _sc_edge()(ewb, hws, hwr, senders, receivers, edge_mask)

    h_new = _node_update(h, agg2, W_node[:_DF], W_node[_DF:], b_node)
    e_new_t = _edge_update(et, m, W_edge[:_DE], W_edge[_DE:], b_edge)
    return h_new, e_new_t.T


# speedup vs baseline: 1.3354x; 1.3354x over previous
import functools

import jax
import jax.numpy as jnp
from jax import lax
from jax.experimental import pallas as pl
from jax.experimental.pallas import tpu as pltpu
from jax.experimental.pallas import tpu_sc as plsc

_N = 10000
_E = 320000
_DF = 128
_DE = 16
_DM = 128


_TE = 12800


def _hw_body(h_ref, ws_ref, wr_ref, hws_ref, hwr_ref):
    h = h_ref[...]
    hws_ref[...] = jnp.dot(h, ws_ref[...], preferred_element_type=jnp.float32)
    hwr_ref[...] = jnp.dot(h, wr_ref[...], preferred_element_type=jnp.float32)


def _node_proj(h, w_s, w_r):
    return pl.pallas_call(
        _hw_body,
        out_shape=[
            jax.ShapeDtypeStruct((_N, _DM), jnp.float32),
            jax.ShapeDtypeStruct((_N, _DM), jnp.float32),
        ],
    )(h, w_s, w_r)


def _ewb_body(et_ref, we_ref, b_ref, out_ref):
    out_ref[...] = (
        lax.dot_general(et_ref[...], we_ref[...], (((0,), (0,)), ((), ())),
                        preferred_element_type=jnp.float32)
        + b_ref[...]
    )


def _edge_proj(et, w_e, b_msg):
    grid = (_E // _TE,)
    return pl.pallas_call(
        _ewb_body,
        grid=grid,
        in_specs=[
            pl.BlockSpec((_DE, _TE), lambda i: (0, i)),
            pl.BlockSpec((_DE, _DM), lambda i: (0, 0)),
            pl.BlockSpec((1, _DM), lambda i: (0, 0)),
        ],
        out_specs=pl.BlockSpec((_TE, _DM), lambda i: (i, 0)),
        out_shape=jax.ShapeDtypeStruct((_E, _DM), jnp.float32),
    )(et, w_e, b_msg.reshape(1, _DM))


_NC = 2
_NS = 16
_NW = _NC * _NS
_EPW = _E // _NW
_C = 40
_NCH = _EPW // _C
_NP = 10240
_RPS = _NP // _NS
_ZR = 40
_WZR = 128


def _sc_body(ewb_hbm, hws_hbm, hwr_hbm, snd_hbm, rcv_hbm, msk_hbm,
             m_hbm, agg_hbm,
             snd_v, rcv_v, msk_v, ewb_v, gs_v, gr_v, mout_v,
             mexp_v, agg_sh, isem, dsem, stsem):
    cid = lax.axis_index("c")
    sid = lax.axis_index("s")
    wid = sid * _NC + cid
    wbase = wid * _EPW

    def zrow(i, _):
        for j in range(_DM // 16):
            mout_v[0][i, pl.ds(j * 16, 16)] = jnp.zeros((16,), jnp.float32)
        return 0

    lax.fori_loop(0, _ZR, zrow, 0)
    for k in range(_RPS // _ZR):
        pltpu.sync_copy(mout_v[0], agg_sh.at[pl.ds(sid * _RPS + k * _ZR, _ZR), :])
    plsc.subcore_barrier()

    def issue_idx(c, s4):
        base = wbase + c * _C
        pltpu.async_copy(snd_hbm.at[pl.ds(base, _C)], snd_v[s4], isem[s4])
        pltpu.async_copy(rcv_hbm.at[pl.ds(base, _C)], rcv_v[s4], isem[s4])

    def drain_idx(s4):
        pltpu.make_async_copy(snd_hbm.at[pl.ds(0, _C)], snd_v[s4], isem[s4]).wait()
        pltpu.make_async_copy(rcv_hbm.at[pl.ds(0, _C)], rcv_v[s4], isem[s4]).wait()

    def issue_data(c, s2, s4):
        base = wbase + c * _C
        pltpu.async_copy(msk_hbm.at[pl.ds(base, _C)], msk_v[s2], dsem[s2])
        pltpu.async_copy(ewb_hbm.at[pl.ds(base, _C), :], ewb_v[s2], dsem[s2])
        pltpu.async_copy(hws_hbm.at[snd_v[s4]], gs_v[s2], dsem[s2])
        pltpu.async_copy(hwr_hbm.at[rcv_v[s4]], gr_v[s2], dsem[s2])

    def drain_data(s2, s4):
        pltpu.make_async_copy(msk_hbm.at[pl.ds(0, _C)], msk_v[s2], dsem[s2]).wait()
        pltpu.make_async_copy(ewb_hbm.at[pl.ds(0, _C), :], ewb_v[s2], dsem[s2]).wait()
        pltpu.make_async_copy(hws_hbm.at[snd_v[s4]], gs_v[s2], dsem[s2]).wait()
        pltpu.make_async_copy(hwr_hbm.at[rcv_v[s4]], gr_v[s2], dsem[s2]).wait()

    def issue_wb(c, s2, s4):
        base = wbase + c * _C
        pltpu.async_copy(mout_v[s2], m_hbm.at[pl.ds(base, _C), :], stsem[s2])
        pltpu.sync_copy(mout_v[s2], agg_sh.at[rcv_v[s4]], add=True)

    def drain_wb(s2):
        pltpu.make_async_copy(mout_v[s2], m_hbm.at[pl.ds(0, _C), :], stsem[s2]).wait()

    def compute(s2, s4):
        mskv = msk_v[s2][:]
        for rs in range(_C):
            mexp_v[pl.ds(rs * 16, 16)] = jnp.take(
                mskv, jnp.full((16,), rs, jnp.int32))

        def row(r, _):
            mv = mexp_v[pl.ds(r * 16, 16)]
            for j in range(_DM // 16):
                sl = pl.ds(j * 16, 16)
                x = ewb_v[s2][r, sl] + gs_v[s2][r, sl] + gr_v[s2][r, sl]
                mout_v[s2][r, sl] = jnp.maximum(x, 0.0) * mv
            return 0

        lax.fori_loop(0, _C, row, 0)

    def step(k, s4, idx_issue=True, data_issue=True, wb_drain=True):
        s2 = s4 % 2
        if idx_issue:
            issue_idx(k + 2, (s4 + 2) % 4)
        if data_issue:
            drain_idx((s4 + 1) % 4)
            issue_data(k + 1, (s4 + 1) % 2, (s4 + 1) % 4)
        drain_data(s2, s4)
        if wb_drain:
            drain_wb(s2)
        compute(s2, s4)
        issue_wb(k, s2, s4)

    issue_idx(jnp.int32(0), 0)
    issue_idx(jnp.int32(1), 1)
    drain_idx(0)
    issue_data(jnp.int32(0), 0, 0)
    step(jnp.int32(0), 0, wb_drain=False)
    step(jnp.int32(1), 1, wb_drain=False)

    def group(i, _):
        k0 = 2 + 4 * i
        for u in range(4):
            step(k0 + u, (2 + u) % 4)
        return 0

    lax.fori_loop(0, (_NCH - 6) // 4, group, 0)

    kt = jnp.int32(_NCH - 4)
    for t in range(4):
        k = kt + t
        s4 = (_NCH - 4 + t) % 4
        step(k, s4,
             idx_issue=(_NCH - 4 + t + 2 <= _NCH - 1),
             data_issue=(_NCH - 4 + t + 1 <= _NCH - 1))
    for c in range(_NCH - 2, _NCH):
        drain_wb(c % 2)

    plsc.subcore_barrier()

    for k in range(_RPS // _WZR):
        r0 = sid * _RPS + k * _WZR
        pltpu.sync_copy(agg_sh.at[pl.ds(r0, _WZR), :],
                        agg_hbm.at[cid, pl.ds(r0, _WZR), :])


@functools.lru_cache(maxsize=1)
def _sc_edge():
    return pl.kernel(
        _sc_body,
        out_type=[
            jax.ShapeDtypeStruct((_E, _DM), jnp.float32),
            jax.ShapeDtypeStruct((_NC, _NP, _DM), jnp.float32),
        ],
        mesh=plsc.VectorSubcoreMesh(
            core_axis_name="c", subcore_axis_name="s",
            num_cores=_NC, num_subcores=_NS,
        ),
        scratch_types=[
            [pltpu.VMEM((_C,), jnp.int32) for _ in range(4)],
            [pltpu.VMEM((_C,), jnp.int32) for _ in range(4)],
            [pltpu.VMEM((_C,), jnp.float32) for _ in range(2)],
            [pltpu.VMEM((_C, _DM), jnp.float32) for _ in range(2)],
            [pltpu.VMEM((_C, _DM), jnp.float32) for _ in range(2)],
            [pltpu.VMEM((_C, _DM), jnp.float32) for _ in range(2)],
            [pltpu.VMEM((_C, _DM), jnp.float32) for _ in range(2)],
            pltpu.VMEM((_C * 16,), jnp.float32),
            pltpu.VMEM_SHARED((_NP, _DM), jnp.float32),
            [pltpu.SemaphoreType.DMA for _ in range(4)],
            [pltpu.SemaphoreType.DMA for _ in range(2)],
            [pltpu.SemaphoreType.DMA for _ in range(2)],
        ],
    )


_TN = 2000


def _node_body(h_ref, agg_ref, wn1_ref, wn2_ref, b_ref, out_ref):
    agg = agg_ref[0] + agg_ref[1]
    x = (
        jnp.dot(h_ref[...], wn1_ref[...], preferred_element_type=jnp.float32)
        + jnp.dot(agg, wn2_ref[...], preferred_element_type=jnp.float32)
        + b_ref[...]
    )
    out_ref[...] = jnp.maximum(x, 0.0)


def _node_update(h, agg2, wn1, wn2, b_node):
    grid = (_N // _TN,)
    return pl.pallas_call(
        _node_body,
        grid=grid,
        in_specs=[
            pl.BlockSpec((_TN, _DF), lambda i: (i, 0)),
            pl.BlockSpec((_NC, _TN, _DM), lambda i: (0, i, 0)),
            pl.BlockSpec((_DF, _DF), lambda i: (0, 0)),
            pl.BlockSpec((_DM, _DF), lambda i: (0, 0)),
            pl.BlockSpec((1, _DF), lambda i: (0, 0)),
        ],
        out_specs=pl.BlockSpec((_TN, _DF), lambda i: (i, 0)),
        out_shape=jax.ShapeDtypeStruct((_N, _DF), jnp.float32),
    )(h, agg2, wn1, wn2, b_node.reshape(1, _DF))


def _edge_body(et_ref, m_ref, we1_ref, we2_ref, b_ref, out_ref):
    x = (
        lax.dot_general(we1_ref[...], et_ref[...], (((0,), (0,)), ((), ())),
                        preferred_element_type=jnp.float32)
        + lax.dot_general(we2_ref[...], m_ref[...], (((0,), (1,)), ((), ())),
                          preferred_element_type=jnp.float32)
        + b_ref[...]
    )
    out_ref[...] = jnp.maximum(x, 0.0)


def _edge_update(et, m, we1, we2, b_edge):
    grid = (_E // _TE,)
    return pl.pallas_call(
        _edge_body,
        grid=grid,
        in_specs=[
            pl.BlockSpec((_DE, _TE), lambda i: (0, i)),
            pl.BlockSpec((_TE, _DM), lambda i: (i, 0)),
            pl.BlockSpec((_DE, _DE), lambda i: (0, 0)),
            pl.BlockSpec((_DM, _DE), lambda i: (0, 0)),
            pl.BlockSpec((_DE, 1), lambda i: (0, 0)),
        ],
        out_specs=pl.BlockSpec((_DE, _TE), lambda i: (0, i)),
        out_shape=jax.ShapeDtypeStruct((_DE, _E), jnp.float32),
    )(et, m, we1, we2, b_edge.reshape(_DE, 1))


@jax.jit
def kernel(h, e, senders, receivers, edge_mask,
           W_msg, b_msg, W_node, b_node, W_edge, b_edge):
    w_e = W_msg[:_DE]
    w_s = W_msg[_DE:_DE + _DF]
    w_r = W_msg[_DE + _DF:]

    et = e.T

    hws, hwr = _node_proj(h, w_s, w_r)
    ewb = _edge_proj(et, w_e, b_msg)

    m, agg2 = _sc_edge()(ewb, hws, hwr, senders, receivers, edge_mask)

    h_new = _node_update(h, agg2, W_node[:_DF], W_node[_DF:], b_node)
    e_new_t = _edge_update(et, m, W_edge[:_DE], W_edge[_DE:], b_edge)
    return h_new, e_new_t.T
